# Initial kernel scaffold; baseline (speedup 1.0000x reference)
import jax
import jax.numpy as jnp
from jax.experimental import pallas as pl


def kernel(x, embeds):
    raise NotImplementedError("write your pallas kernel here")


# speedup vs baseline: 1.4273x; 1.4273x over previous
import functools

import jax
import jax.numpy as jnp
from jax import lax
from jax.experimental import pallas as pl
from jax.experimental.pallas import tpu as pltpu
from jax.experimental.pallas import tpu_sc as plsc

_DIM = 32
_NQ = 4
_K = 8192
_TN = 512
_KB = 1024


def _argmin_body(res_ref, qnt_ref, augbf_ref, e2_ref, rout_ref, idx_ref,
                 sq_ref):
    i = pl.program_id(0)

    @pl.when(i == 0)
    def _():
        sq_ref[0, 0] = 0.0

    res = res_ref[...]
    st = res + (qnt_ref[:, :_DIM] - res)
    rsd = res - st
    rout_ref[...] = rsd
    sq_ref[0, 0] += jnp.sum(rsd * rsd)

    rbf = rsd.astype(jnp.bfloat16)

    best_d = jnp.full((_TN,), jnp.inf, jnp.float32)
    best_i = jnp.zeros((_TN,), jnp.int32)
    for kb in range(_K // _KB):
        ksl = pl.ds(kb * _KB, _KB)
        m2 = jnp.dot(rbf, augbf_ref[:, ksl],
                     preferred_element_type=jnp.float32)
        d = e2_ref[:, ksl] + m2
        lmin = jnp.min(d, axis=1)
        io = lax.broadcasted_iota(jnp.int32, (_TN, _KB), 1)
        lidx = jnp.min(jnp.where(d <= lmin[:, None], io, _KB), axis=1)
        upd = lmin < best_d
        best_i = jnp.where(upd, lidx + kb * _KB, best_i)
        best_d = jnp.where(upd, lmin, best_d)
    idx_ref[...] = best_i


def _argmin_call(res_prev, quant_prev, augbf_q, e2_q, n_tok):
    nt = n_tok // _TN
    return pl.pallas_call(
        _argmin_body,
        grid=(nt,),
        in_specs=[
            pl.BlockSpec((_TN, _DIM), lambda i: (i, 0)),
            pl.BlockSpec((_TN, 128), lambda i: (i, 0)),
            pl.BlockSpec((_DIM, _K), lambda i: (0, 0)),
            pl.BlockSpec((1, _K), lambda i: (0, 0)),
        ],
        out_specs=[
            pl.BlockSpec((_TN, _DIM), lambda i: (i, 0)),
            pl.BlockSpec((_TN,), lambda i: (i,)),
            pl.BlockSpec(memory_space=pltpu.SMEM, block_shape=(1, 1),
                         index_map=lambda i: (0, 0)),
        ],
        out_shape=[
            jax.ShapeDtypeStruct((n_tok, _DIM), jnp.float32),
            jax.ShapeDtypeStruct((n_tok,), jnp.int32),
            jax.ShapeDtypeStruct((1, 1), jnp.float32),
        ],
    )(res_prev, quant_prev, augbf_q, e2_q)


def _final_call(xf, res3, quant3, sqs, n_tok):
    nt = n_tok // _TN

    def body(x_ref, res_ref, qnt_ref, sqs_ref, qout_ref, loss_ref):
        i = pl.program_id(0)

        @pl.when(i == 0)
        def _():
            loss_ref[0, 0] = 0.0

        res = res_ref[...]
        st = res + (qnt_ref[:, :_DIM] - res)
        r4 = res - st
        qout_ref[...] = x_ref[...] - r4
        loss_ref[0, 0] += jnp.sum(r4 * r4)

        @pl.when(i == nt - 1)
        def _():
            tot = loss_ref[0, 0] + sqs_ref[0, 0] + sqs_ref[0, 1] + sqs_ref[0, 2]
            loss_ref[0, 0] = 2.0 * tot / jnp.float32(n_tok * _DIM)

    return pl.pallas_call(
        body,
        grid=(nt,),
        in_specs=[
            pl.BlockSpec((_TN, _DIM), lambda i: (i, 0)),
            pl.BlockSpec((_TN, _DIM), lambda i: (i, 0)),
            pl.BlockSpec((_TN, 128), lambda i: (i, 0)),
            pl.BlockSpec(memory_space=pltpu.SMEM, block_shape=(1, 3),
                         index_map=lambda i: (0, 0)),
        ],
        out_specs=[
            pl.BlockSpec((_TN, _DIM), lambda i: (i, 0)),
            pl.BlockSpec(memory_space=pltpu.SMEM, block_shape=(1, 1),
                         index_map=lambda i: (0, 0)),
        ],
        out_shape=[
            jax.ShapeDtypeStruct((n_tok, _DIM), jnp.float32),
            jax.ShapeDtypeStruct((1, 1), jnp.float32),
        ],
    )(xf, res3, quant3, sqs)


def _make_sc_gather(n_tok):
    info = plsc.get_sparse_core_info()
    nc, ns = info.num_cores, info.num_subcores
    nw = nc * ns
    b_per_w = n_tok // nw
    mesh = plsc.VectorSubcoreMesh(core_axis_name="c", subcore_axis_name="s")

    @functools.partial(
        pl.kernel, mesh=mesh,
        out_type=jax.ShapeDtypeStruct((n_tok, 128), jnp.float32),
        scratch_types=[
            pltpu.VMEM((b_per_w,), jnp.int32),
            pltpu.VMEM((b_per_w, 128), jnp.float32),
            pltpu.SemaphoreType.DMA,
        ],
    )
    def gather(table_hbm, idx_hbm, out_hbm, idx_v, rows_v, sem):
        wid = lax.axis_index("s") * nc + lax.axis_index("c")
        base = wid * b_per_w
        pltpu.sync_copy(idx_hbm.at[pl.ds(base, b_per_w)], idx_v)
        pltpu.async_copy(table_hbm.at[idx_v], rows_v, sem).wait()
        pltpu.sync_copy(rows_v, out_hbm.at[pl.ds(base, b_per_w)])

    return gather


def kernel(x, embeds):
    b, s, _ = x.shape
    n_tok = b * s
    xf = x.reshape(n_tok, _DIM)

    augbf = (-2.0 * jnp.transpose(embeds, (0, 2, 1))).astype(jnp.bfloat16)
    e2 = jnp.sum(embeds ** 2, axis=2)[:, None, :]

    sc_gather = _make_sc_gather(n_tok)
    emb_rnd = embeds.astype(jnp.bfloat16).astype(jnp.float32)
    emb_pad = jnp.pad(emb_rnd, ((0, 0), (0, 0), (0, 128 - _DIM)))

    zeros = jnp.zeros((n_tok, 128), jnp.float32)
    res_prev, quant_prev = xf, zeros
    idxs, sqs = [], []
    for q in range(_NQ):
        res_q, idx_q, sq_q = _argmin_call(res_prev, quant_prev, augbf[q],
                                          e2[q], n_tok)
        quant_q = sc_gather(emb_pad[q], idx_q)
        idxs.append(idx_q)
        sqs.append(sq_q)
        res_prev, quant_prev = res_q, quant_q

    sq123 = jnp.concatenate([sqs[1], sqs[2], sqs[3]], axis=1)
    qout, loss = _final_call(xf, res_prev, quant_prev, sq123, n_tok)

    quantized_out = qout.reshape(b, s, _DIM)
    all_indices = jnp.stack(idxs, axis=-1).reshape(b, s, _NQ)
    return (quantized_out, all_indices, loss[0, 0])


# speedup vs baseline: 1.7074x; 1.1963x over previous
import functools

import jax
import jax.numpy as jnp
from jax import lax
from jax.experimental import pallas as pl
from jax.experimental.pallas import tpu as pltpu
from jax.experimental.pallas import tpu_sc as plsc

_DIM = 32
_NQ = 4
_K = 8192
_TN = 512
_KB = 1024


def _argmin_body(res_ref, qnt_ref, augbf_ref, e2_ref, rout_ref, idx_ref,
                 sq_ref):
    i = pl.program_id(0)

    @pl.when(i == 0)
    def _():
        sq_ref[0, 0] = 0.0

    res = res_ref[...]
    st = res + (qnt_ref[:, :_DIM] - res)
    rsd = res - st
    rout_ref[...] = rsd
    sq_ref[0, 0] += jnp.sum(rsd * rsd)

    rbf = rsd.astype(jnp.bfloat16)

    macc = jnp.full((_TN, _KB), jnp.inf, jnp.float32)
    bacc = jnp.zeros((_TN, _KB), jnp.float32)
    for kb in range(_K // _KB):
        ksl = pl.ds(kb * _KB, _KB)
        m2 = jnp.dot(rbf, augbf_ref[:, ksl],
                     preferred_element_type=jnp.float32)
        d = e2_ref[:, ksl] + m2
        upd = d < macc
        macc = jnp.where(upd, d, macc)
        bacc = jnp.where(upd, jnp.float32(kb), bacc)
    lmin = jnp.min(macc, axis=1)
    io = lax.broadcasted_iota(jnp.int32, (_TN, _KB), 1).astype(jnp.float32)
    pk = bacc * jnp.float32(_KB) + io
    cand = jnp.where(macc <= lmin[:, None], pk, jnp.float32(2 * _K))
    idx_ref[...] = jnp.min(cand, axis=1).astype(jnp.int32)


def _argmin_call(res_prev, quant_prev, augbf_q, e2_q, n_tok):
    nt = n_tok // _TN
    return pl.pallas_call(
        _argmin_body,
        grid=(nt,),
        in_specs=[
            pl.BlockSpec((_TN, _DIM), lambda i: (i, 0)),
            pl.BlockSpec((_TN, 128), lambda i: (i, 0)),
            pl.BlockSpec((_DIM, _K), lambda i: (0, 0)),
            pl.BlockSpec((1, _K), lambda i: (0, 0)),
        ],
        out_specs=[
            pl.BlockSpec((_TN, _DIM), lambda i: (i, 0)),
            pl.BlockSpec((_TN,), lambda i: (i,)),
            pl.BlockSpec(memory_space=pltpu.SMEM, block_shape=(1, 1),
                         index_map=lambda i: (0, 0)),
        ],
        out_shape=[
            jax.ShapeDtypeStruct((n_tok, _DIM), jnp.float32),
            jax.ShapeDtypeStruct((n_tok,), jnp.int32),
            jax.ShapeDtypeStruct((1, 1), jnp.float32),
        ],
    )(res_prev, quant_prev, augbf_q, e2_q)


def _final_call(xf, res3, quant3, sqs, n_tok):
    nt = n_tok // _TN

    def body(x_ref, res_ref, qnt_ref, sqs_ref, qout_ref, loss_ref):
        i = pl.program_id(0)

        @pl.when(i == 0)
        def _():
            loss_ref[0, 0] = 0.0

        res = res_ref[...]
        st = res + (qnt_ref[:, :_DIM] - res)
        r4 = res - st
        qout_ref[...] = x_ref[...] - r4
        loss_ref[0, 0] += jnp.sum(r4 * r4)

        @pl.when(i == nt - 1)
        def _():
            tot = loss_ref[0, 0] + sqs_ref[0, 0] + sqs_ref[0, 1] + sqs_ref[0, 2]
            loss_ref[0, 0] = 2.0 * tot / jnp.float32(n_tok * _DIM)

    return pl.pallas_call(
        body,
        grid=(nt,),
        in_specs=[
            pl.BlockSpec((_TN, _DIM), lambda i: (i, 0)),
            pl.BlockSpec((_TN, _DIM), lambda i: (i, 0)),
            pl.BlockSpec((_TN, 128), lambda i: (i, 0)),
            pl.BlockSpec(memory_space=pltpu.SMEM, block_shape=(1, 3),
                         index_map=lambda i: (0, 0)),
        ],
        out_specs=[
            pl.BlockSpec((_TN, _DIM), lambda i: (i, 0)),
            pl.BlockSpec(memory_space=pltpu.SMEM, block_shape=(1, 1),
                         index_map=lambda i: (0, 0)),
        ],
        out_shape=[
            jax.ShapeDtypeStruct((n_tok, _DIM), jnp.float32),
            jax.ShapeDtypeStruct((1, 1), jnp.float32),
        ],
    )(xf, res3, quant3, sqs)


def _make_sc_gather(n_tok):
    info = plsc.get_sparse_core_info()
    nc, ns = info.num_cores, info.num_subcores
    nw = nc * ns
    b_per_w = n_tok // nw
    mesh = plsc.VectorSubcoreMesh(core_axis_name="c", subcore_axis_name="s")

    @functools.partial(
        pl.kernel, mesh=mesh,
        out_type=jax.ShapeDtypeStruct((n_tok, 128), jnp.float32),
        scratch_types=[
            pltpu.VMEM((b_per_w,), jnp.int32),
            pltpu.VMEM((b_per_w, 128), jnp.float32),
            pltpu.SemaphoreType.DMA,
        ],
    )
    def gather(table_hbm, idx_hbm, out_hbm, idx_v, rows_v, sem):
        wid = lax.axis_index("s") * nc + lax.axis_index("c")
        base = wid * b_per_w
        pltpu.sync_copy(idx_hbm.at[pl.ds(base, b_per_w)], idx_v)
        pltpu.async_copy(table_hbm.at[idx_v], rows_v, sem).wait()
        pltpu.sync_copy(rows_v, out_hbm.at[pl.ds(base, b_per_w)])

    return gather


def kernel(x, embeds):
    b, s, _ = x.shape
    n_tok = b * s
    xf = x.reshape(n_tok, _DIM)

    augbf = (-2.0 * jnp.transpose(embeds, (0, 2, 1))).astype(jnp.bfloat16)
    e2 = jnp.sum(embeds ** 2, axis=2)[:, None, :]

    sc_gather = _make_sc_gather(n_tok)
    emb_rnd = embeds.astype(jnp.bfloat16).astype(jnp.float32)
    emb_pad = jnp.pad(emb_rnd, ((0, 0), (0, 0), (0, 128 - _DIM)))

    zeros = jnp.zeros((n_tok, 128), jnp.float32)
    res_prev, quant_prev = xf, zeros
    idxs, sqs = [], []
    for q in range(_NQ):
        res_q, idx_q, sq_q = _argmin_call(res_prev, quant_prev, augbf[q],
                                          e2[q], n_tok)
        quant_q = sc_gather(emb_pad[q], idx_q)
        idxs.append(idx_q)
        sqs.append(sq_q)
        res_prev, quant_prev = res_q, quant_q

    sq123 = jnp.concatenate([sqs[1], sqs[2], sqs[3]], axis=1)
    qout, loss = _final_call(xf, res_prev, quant_prev, sq123, n_tok)

    quantized_out = qout.reshape(b, s, _DIM)
    all_indices = jnp.stack(idxs, axis=-1).reshape(b, s, _NQ)
    return (quantized_out, all_indices, loss[0, 0])
